# Initial kernel scaffold; baseline (speedup 1.0000x reference)
import jax
import jax.numpy as jnp
from jax.experimental import pallas as pl


def kernel(x, edge_index, W1, a_s1, a_d1, b1, W2, a_s2, a_d2, b2, Wl, bl):
    raise NotImplementedError("write your pallas kernel here")


# speedup vs baseline: 62.2577x; 62.2577x over previous
import jax
import jax.numpy as jnp
from jax import lax
from jax.experimental import pallas as pl
from jax.experimental.pallas import tpu as pltpu
from jax.experimental.pallas import tpu_sc as plsc

N = 50000
E = 800000
ER = E + N
NW = 32
KL = 1024
KC = 1024
KD = 512
EP = KL * NW * (-(-ER // (KL * NW)))
_SC_PARAMS = dict(needs_layout_passes=False, use_tc_tiling_on_sc=False)


def _i16(v):
    return jnp.full((16,), v, jnp.int32)


def _leaky(v):
    return jnp.where(v > 0, v, 0.2 * v)


def _dense1_body(x_ref, w1_ref, ats_ref, atd_ref, s1_ref, d1_ref, mx_ref):
    i = pl.program_id(0)
    xb = x_ref[...]
    h1 = jnp.dot(xb, w1_ref[...], preferred_element_type=jnp.float32)
    acols, dcols = [], []
    for h in range(4):
        hh = h1[:, 32 * h:32 * h + 32]
        acols.append(jnp.sum(hh * ats_ref[h:h + 1, :], axis=1, keepdims=True))
        dcols.append(jnp.sum(hh * atd_ref[h:h + 1, :], axis=1, keepdims=True))
    ts = jnp.concatenate(acols + [xb], axis=1)
    td = jnp.concatenate(dcols + [jnp.zeros_like(xb)], axis=1)
    s1_ref[...] = ts
    d1_ref[...] = td
    m8 = jnp.max(ts[:, :8], axis=0, keepdims=True)
    m = jnp.concatenate([m8, m8], axis=1)

    @pl.when(i == 0)
    def _():
        mx_ref[...] = m

    @pl.when(i > 0)
    def _():
        mx_ref[...] = jnp.maximum(mx_ref[...], m)


def _dense1(x, w1, ats, atd):
    B = 5000
    return pl.pallas_call(
        _dense1_body,
        grid=(N // B,),
        in_specs=[
            pl.BlockSpec((B, 4), lambda i: (i, 0)),
            pl.BlockSpec((4, 128), lambda i: (0, 0)),
            pl.BlockSpec((4, 32), lambda i: (0, 0)),
            pl.BlockSpec((4, 32), lambda i: (0, 0)),
        ],
        out_specs=[
            pl.BlockSpec((B, 8), lambda i: (i, 0)),
            pl.BlockSpec((B, 8), lambda i: (i, 0)),
            pl.BlockSpec((1, 16), lambda i: (0, 0)),
        ],
        out_shape=[
            jax.ShapeDtypeStruct((N, 8), jnp.float32),
            jax.ShapeDtypeStruct((N, 8), jnp.float32),
            jax.ShapeDtypeStruct((1, 16), jnp.float32),
        ],
    )(x, w1, ats, atd)


def _l1_body(s1_h, d1_h, src_h, dst_h, mx_h, zu_h, up_h,
             sidx, didx, srows, drows, ubuf, mxv, ush, sem_a, sem_b):
    cid = lax.axis_index("c")
    sid = lax.axis_index("s")
    wid = cid * 16 + sid

    @pl.when(sid == 0)
    def _():
        pltpu.sync_copy(zu_h, ush)

    pltpu.sync_copy(mx_h, mxv)
    plsc.subcore_barrier()

    iota = lax.iota(jnp.int32, 16)
    mx_splat = [mxv[h] for h in range(4)]

    def zpad(g, c):
        rows = g * 16 + iota
        zv = jnp.zeros((16,), jnp.float32)
        for cc in range(20, 24):
            plsc.store_scatter(ubuf, [rows, _i16(cc)], zv)
        return c

    lax.fori_loop(0, KL // 16, zpad, 0)

    def chunk(c, carry):
        base = (wid * (EP // (KL * NW)) + c) * KL
        pltpu.sync_copy(src_h.at[pl.ds(base, KL)], sidx)
        pltpu.sync_copy(dst_h.at[pl.ds(base, KL)], didx)
        d1 = pltpu.async_copy(s1_h.at[sidx], srows, sem_a)
        d2 = pltpu.async_copy(d1_h.at[didx], drows, sem_b)
        d1.wait()
        d2.wait()

        def group(g, cc):
            e0 = g * 16
            rows = e0 + iota
            asv = [plsc.load_gather(srows, [rows, _i16(h)]) for h in range(4)]
            xv = [plsc.load_gather(srows, [rows, _i16(4 + k)]) for k in range(4)]
            adv = [plsc.load_gather(drows, [rows, _i16(h)]) for h in range(4)]
            valid = (base + e0 + iota) < ER
            for h in range(4):
                a = _leaky(asv[h] + adv[h])
                cshift = _leaky(mx_splat[h] + adv[h])
                w = jnp.exp(a - cshift)
                w = jnp.where(valid, w, 0.0)
                plsc.store_scatter(ubuf, [rows, _i16(h)], w)
                for k in range(4):
                    plsc.store_scatter(ubuf, [rows, _i16(4 + h * 4 + k)], w * xv[k])
            return cc

        lax.fori_loop(0, KL // 16, group, 0)
        pltpu.sync_copy(ubuf, ush.at[didx], add=True)
        return carry

    lax.fori_loop(0, EP // (KL * NW), chunk, 0)
    plsc.subcore_barrier()

    @pl.when(sid == 0)
    def _():
        pltpu.sync_copy(ush, up_h.at[cid])


def _l1_edges(s1, d1, srcp, dstp, mx1, zu):
    mesh = plsc.VectorSubcoreMesh(core_axis_name="c", subcore_axis_name="s")
    f = pl.kernel(
        _l1_body,
        out_type=jax.ShapeDtypeStruct((2, N, 24), jnp.float32),
        mesh=mesh,
        scratch_types=[
            pltpu.VMEM((KL,), jnp.int32),
            pltpu.VMEM((KL,), jnp.int32),
            pltpu.VMEM((KL, 8), jnp.float32),
            pltpu.VMEM((KL, 8), jnp.float32),
            pltpu.VMEM((KL, 24), jnp.float32),
            pltpu.VMEM((4, 16), jnp.float32),
            pltpu.VMEM_SHARED((N, 24), jnp.float32),
            pltpu.SemaphoreType.DMA,
            pltpu.SemaphoreType.DMA,
        ],
        compiler_params=pltpu.CompilerParams(**_SC_PARAMS),
    )
    return f(s1, d1, srcp, dstp, mx1, zu)


def _dense2_body(ua_ref, ub_ref, w1h0, w1h1, w1h2, w1h3, b1_ref, w2c_ref,
                 ats2_ref, atd2_ref, h2_ref, as2_ref, ad2_ref, mx2_ref):
    i = pl.program_id(0)
    u = ua_ref[...] + ub_ref[...]
    outs = []
    for h, w1h in enumerate((w1h0, w1h1, w1h2, w1h3)):
        den = u[:, h:h + 1]
        den = jnp.where(den > 0, den, 1.0)
        th = u[:, 4 + 4 * h:8 + 4 * h] / den
        outs.append(jnp.dot(th, w1h[...], preferred_element_type=jnp.float32))
    o1 = jnp.concatenate(outs, axis=1) + b1_ref[...]
    x1 = jnp.where(o1 > 0, o1, jnp.exp(o1) - 1.0)
    r2 = jnp.dot(x1, w2c_ref[...], preferred_element_type=jnp.float32)
    as2 = jnp.sum(r2 * ats2_ref[...], axis=1, keepdims=True)
    ad2 = jnp.sum(r2 * atd2_ref[...], axis=1, keepdims=True)
    h2_ref[...] = r2
    as2_ref[...] = as2
    ad2_ref[...] = ad2
    m = jnp.max(as2, axis=0, keepdims=True)

    @pl.when(i == 0)
    def _():
        mx2_ref[...] = m

    @pl.when(i > 0)
    def _():
        mx2_ref[...] = jnp.maximum(mx2_ref[...], m)


def _dense2(ua, ub, w1heads, b1r, w2c, ats2, atd2):
    B = 2000
    return pl.pallas_call(
        _dense2_body,
        grid=(N // B,),
        in_specs=[
            pl.BlockSpec((B, 24), lambda i: (i, 0)),
            pl.BlockSpec((B, 24), lambda i: (i, 0)),
            pl.BlockSpec((4, 32), lambda i: (0, 0)),
            pl.BlockSpec((4, 32), lambda i: (0, 0)),
            pl.BlockSpec((4, 32), lambda i: (0, 0)),
            pl.BlockSpec((4, 32), lambda i: (0, 0)),
            pl.BlockSpec((1, 128), lambda i: (0, 0)),
            pl.BlockSpec((128, 32), lambda i: (0, 0)),
            pl.BlockSpec((1, 32), lambda i: (0, 0)),
            pl.BlockSpec((1, 32), lambda i: (0, 0)),
        ],
        out_specs=[
            pl.BlockSpec((B, 32), lambda i: (i, 0)),
            pl.BlockSpec((B, 1), lambda i: (i, 0)),
            pl.BlockSpec((B, 1), lambda i: (i, 0)),
            pl.BlockSpec((1, 1), lambda i: (0, 0)),
        ],
        out_shape=[
            jax.ShapeDtypeStruct((N, 32), jnp.float32),
            jax.ShapeDtypeStruct((N, 1), jnp.float32),
            jax.ShapeDtypeStruct((N, 1), jnp.float32),
            jax.ShapeDtypeStruct((1, 1), jnp.float32),
        ],
    )(ua, ub, *w1heads, b1r, w2c, ats2, atd2)


def _pc_body(src_h, dst_h, as2_h, ad2_h, mx2_h, zn_h, w2_h, dena_h, denb_h,
             sidx, didx, asv, adv, wbuf, mxv, dsh, sem_a, sem_b):
    cid = lax.axis_index("c")
    sid = lax.axis_index("s")
    wid = cid * 16 + sid

    @pl.when(sid == 0)
    def _():
        pltpu.sync_copy(zn_h, dsh)

    pltpu.sync_copy(mx2_h, mxv)
    plsc.subcore_barrier()

    iota = lax.iota(jnp.int32, 16)
    mxs = mxv[...]

    def chunk(c, carry):
        base = (wid * (EP // (KC * NW)) + c) * KC
        pltpu.sync_copy(src_h.at[pl.ds(base, KC)], sidx)
        pltpu.sync_copy(dst_h.at[pl.ds(base, KC)], didx)
        d1 = pltpu.async_copy(as2_h.at[sidx], asv, sem_a)
        d2 = pltpu.async_copy(ad2_h.at[didx], adv, sem_b)
        d1.wait()
        d2.wait()

        def group(g, cc):
            rows = g * 16 + iota
            a_s = plsc.load_gather(asv, [rows])
            a_d = plsc.load_gather(adv, [rows])
            a = _leaky(a_s + a_d)
            cshift = _leaky(mxs + a_d)
            w = jnp.exp(a - cshift)
            valid = (base + g * 16 + iota) < ER
            w = jnp.where(valid, w, 0.0)
            plsc.store_scatter(wbuf, [rows], w)
            return cc

        lax.fori_loop(0, KC // 16, group, 0)
        pltpu.sync_copy(wbuf, w2_h.at[pl.ds(base, KC)])
        pltpu.sync_copy(wbuf, dsh.at[didx], add=True)
        return carry

    lax.fori_loop(0, EP // (KC * NW), chunk, 0)
    plsc.subcore_barrier()

    @pl.when(jnp.logical_and(sid == 0, cid == 0))
    def _():
        pltpu.sync_copy(dsh, dena_h)

    @pl.when(jnp.logical_and(sid == 0, cid == 1))
    def _():
        pltpu.sync_copy(dsh, denb_h)


def _pass_c(srcp, dstp, as2, ad2, mx2, zn):
    mesh = plsc.VectorSubcoreMesh(core_axis_name="c", subcore_axis_name="s")
    f = pl.kernel(
        _pc_body,
        out_type=(
            jax.ShapeDtypeStruct((EP,), jnp.float32),
            jax.ShapeDtypeStruct((N,), jnp.float32),
            jax.ShapeDtypeStruct((N,), jnp.float32),
        ),
        mesh=mesh,
        scratch_types=[
            pltpu.VMEM((KC,), jnp.int32),
            pltpu.VMEM((KC,), jnp.int32),
            pltpu.VMEM((KC,), jnp.float32),
            pltpu.VMEM((KC,), jnp.float32),
            pltpu.VMEM((KC,), jnp.float32),
            pltpu.VMEM((16,), jnp.float32),
            pltpu.VMEM_SHARED((N,), jnp.float32),
            pltpu.SemaphoreType.DMA,
            pltpu.SemaphoreType.DMA,
        ],
        compiler_params=pltpu.CompilerParams(**_SC_PARAMS),
    )
    return f(srcp, dstp, as2, ad2, mx2, zn)


def _pd_body(src_h, dst_h, w2_h, dena_h, denb_h, h2_h, zo_h, attn_h, o2_h,
             sidx, didx, wv, dav, dbv, hrows, attnb, osh,
             sem_a, sem_b, sem_c):
    cid = lax.axis_index("c")
    sid = lax.axis_index("s")
    wid = cid * 16 + sid

    @pl.when(sid == 0)
    def _():
        pltpu.sync_copy(zo_h, osh)

    plsc.subcore_barrier()

    iota = lax.iota(jnp.int32, 16)

    def chunk(c, carry):
        base = (wid * (EP // (KD * NW)) + c) * KD
        pltpu.sync_copy(src_h.at[pl.ds(base, KD)], sidx)
        pltpu.sync_copy(dst_h.at[pl.ds(base, KD)], didx)
        pltpu.sync_copy(w2_h.at[pl.ds(base, KD)], wv)
        d1 = pltpu.async_copy(dena_h.at[didx], dav, sem_a)
        d2 = pltpu.async_copy(denb_h.at[didx], dbv, sem_b)
        d3 = pltpu.async_copy(h2_h.at[sidx], hrows, sem_c)
        d1.wait()
        d2.wait()
        d3.wait()

        def group(g, cc):
            rows = g * 16 + iota
            w = plsc.load_gather(wv, [rows])
            da = plsc.load_gather(dav, [rows])
            db = plsc.load_gather(dbv, [rows])
            att = w / (da + db + 1e-16)
            plsc.store_scatter(attnb, [rows], att)
            for col in range(32):
                hv = plsc.load_gather(hrows, [rows, _i16(col)])
                plsc.store_scatter(hrows, [rows, _i16(col)], hv * att)
            return cc

        lax.fori_loop(0, KD // 16, group, 0)
        pltpu.sync_copy(attnb, attn_h.at[pl.ds(base, KD)])
        pltpu.sync_copy(hrows, osh.at[didx], add=True)
        return carry

    lax.fori_loop(0, EP // (KD * NW), chunk, 0)
    plsc.subcore_barrier()

    @pl.when(sid == 0)
    def _():
        pltpu.sync_copy(osh, o2_h.at[cid])


def _pass_d(srcp, dstp, w2, dena, denb, h2, zo):
    mesh = plsc.VectorSubcoreMesh(core_axis_name="c", subcore_axis_name="s")
    f = pl.kernel(
        _pd_body,
        out_type=(
            jax.ShapeDtypeStruct((EP,), jnp.float32),
            jax.ShapeDtypeStruct((2, N, 32), jnp.float32),
        ),
        mesh=mesh,
        scratch_types=[
            pltpu.VMEM((KD,), jnp.int32),
            pltpu.VMEM((KD,), jnp.int32),
            pltpu.VMEM((KD,), jnp.float32),
            pltpu.VMEM((KD,), jnp.float32),
            pltpu.VMEM((KD,), jnp.float32),
            pltpu.VMEM((KD, 32), jnp.float32),
            pltpu.VMEM((KD,), jnp.float32),
            pltpu.VMEM_SHARED((N, 32), jnp.float32),
            pltpu.SemaphoreType.DMA,
            pltpu.SemaphoreType.DMA,
            pltpu.SemaphoreType.DMA,
        ],
        compiler_params=pltpu.CompilerParams(**_SC_PARAMS),
    )
    return f(srcp, dstp, w2, dena, denb, h2, zo)


def _dense3_body(oa_ref, ob_ref, b2_ref, wl_ref, bl_ref, sum_ref, log_ref):
    i = pl.program_id(0)
    o = oa_ref[...] + ob_ref[...] + b2_ref[...]
    x2 = jnp.where(o > 0, o, jnp.exp(o) - 1.0)
    s = jnp.sum(x2, axis=0, keepdims=True)

    @pl.when(i == 0)
    def _():
        sum_ref[...] = s

    @pl.when(i > 0)
    def _():
        sum_ref[...] = sum_ref[...] + s

    @pl.when(i == pl.num_programs(0) - 1)
    def _():
        log_ref[...] = (
            jnp.dot(sum_ref[...] * (1.0 / N), wl_ref[...],
                    preferred_element_type=jnp.float32) + bl_ref[...])


def _dense3(oa, ob, b2r, wl, blr):
    B = 2000
    return pl.pallas_call(
        _dense3_body,
        grid=(N // B,),
        in_specs=[
            pl.BlockSpec((B, 32), lambda i: (i, 0)),
            pl.BlockSpec((B, 32), lambda i: (i, 0)),
            pl.BlockSpec((1, 32), lambda i: (0, 0)),
            pl.BlockSpec((32, 2), lambda i: (0, 0)),
            pl.BlockSpec((1, 2), lambda i: (0, 0)),
        ],
        out_specs=[
            pl.BlockSpec((1, 32), lambda i: (0, 0)),
            pl.BlockSpec((1, 2), lambda i: (0, 0)),
        ],
        out_shape=[
            jax.ShapeDtypeStruct((1, 32), jnp.float32),
            jax.ShapeDtypeStruct((1, 2), jnp.float32),
        ],
    )(oa, ob, b2r, wl, blr)


def kernel(x, edge_index, W1, a_s1, a_d1, b1, W2, a_s2, a_d2, b2, Wl, bl):
    loops = jnp.arange(N, dtype=jnp.int32)
    padz = jnp.zeros((EP - ER,), jnp.int32)
    srcp = jnp.concatenate([edge_index[0].astype(jnp.int32), loops, padz])
    dstp = jnp.concatenate([edge_index[1].astype(jnp.int32), loops, padz])

    w1r = W1.reshape(4, 4, 32)
    w1heads = [w1r[:, h, :] for h in range(4)]

    zu = jnp.zeros((N, 24), jnp.float32)
    zn = jnp.zeros((N,), jnp.float32)
    zo = jnp.zeros((N, 32), jnp.float32)

    s1, d1t, mx1 = _dense1(x, W1, a_s1, a_d1)
    mxtab = jnp.broadcast_to(mx1.reshape(16)[:4][:, None], (4, 16))
    up = _l1_edges(s1, d1t, srcp, dstp, mxtab, zu)
    h2, as2, ad2, mx2 = _dense2(up[0], up[1], w1heads, b1.reshape(1, 128), W2,
                                a_s2.reshape(1, 32), a_d2.reshape(1, 32))
    mx2v = jnp.broadcast_to(mx2.reshape(1), (16,))
    w2e, dena, denb = _pass_c(srcp, dstp, as2.reshape(N), ad2.reshape(N),
                              mx2v, zn)
    attnp, o2 = _pass_d(srcp, dstp, w2e, dena, denb, h2, zo)
    _, logits = _dense3(o2[0], o2[1], b2.reshape(1, 32), Wl, bl.reshape(1, 2))
    attn = attnp[:ER].reshape(ER, 1)
    return logits, attn


# speedup vs baseline: 66.4771x; 1.0678x over previous
import jax
import jax.numpy as jnp
from jax import lax
from jax.experimental import pallas as pl
from jax.experimental.pallas import tpu as pltpu
from jax.experimental.pallas import tpu_sc as plsc

N = 50000
E = 800000
ER = E + N
NW = 32
KL = 512
KC = 1024
KD = 256
EP = 1024 * NW * (-(-ER // (1024 * NW)))
_SC_PARAMS = dict(needs_layout_passes=False, use_tc_tiling_on_sc=False)


def _i16(v):
    return jnp.full((16,), v, jnp.int32)


def _leaky(v):
    return jnp.where(v > 0, v, 0.2 * v)


def _dense1_body(x_ref, w1_ref, ats_ref, atd_ref, s1_ref, d1_ref, mx_ref):
    i = pl.program_id(0)
    xb = x_ref[...]
    h1 = jnp.dot(xb, w1_ref[...], preferred_element_type=jnp.float32)
    acols, dcols = [], []
    for h in range(4):
        hh = h1[:, 32 * h:32 * h + 32]
        acols.append(jnp.sum(hh * ats_ref[h:h + 1, :], axis=1, keepdims=True))
        dcols.append(jnp.sum(hh * atd_ref[h:h + 1, :], axis=1, keepdims=True))
    ts = jnp.concatenate(acols + [xb], axis=1)
    td = jnp.concatenate(dcols + [jnp.zeros_like(xb)], axis=1)
    s1_ref[...] = ts
    d1_ref[...] = td
    m8 = jnp.max(ts[:, :8], axis=0, keepdims=True)
    m = jnp.concatenate([m8, m8], axis=1)

    @pl.when(i == 0)
    def _():
        mx_ref[...] = m

    @pl.when(i > 0)
    def _():
        mx_ref[...] = jnp.maximum(mx_ref[...], m)


def _dense1(x, w1, ats, atd):
    B = 5000
    return pl.pallas_call(
        _dense1_body,
        grid=(N // B,),
        in_specs=[
            pl.BlockSpec((B, 4), lambda i: (i, 0)),
            pl.BlockSpec((4, 128), lambda i: (0, 0)),
            pl.BlockSpec((4, 32), lambda i: (0, 0)),
            pl.BlockSpec((4, 32), lambda i: (0, 0)),
        ],
        out_specs=[
            pl.BlockSpec((B, 8), lambda i: (i, 0)),
            pl.BlockSpec((B, 8), lambda i: (i, 0)),
            pl.BlockSpec((1, 16), lambda i: (0, 0)),
        ],
        out_shape=[
            jax.ShapeDtypeStruct((N, 8), jnp.float32),
            jax.ShapeDtypeStruct((N, 8), jnp.float32),
            jax.ShapeDtypeStruct((1, 16), jnp.float32),
        ],
    )(x, w1, ats, atd)


def _l1_body(s1_h, d1_h, src_h, dst_h, mx_h, zu_h, up_h,
             sidx, didx, srows, drows, ubuf, mxv, ush, sem_a, sem_b):
    cid = lax.axis_index("c")
    sid = lax.axis_index("s")
    wid = cid * 16 + sid

    @pl.when(sid == 0)
    def _():
        pltpu.sync_copy(zu_h, ush)

    pltpu.sync_copy(mx_h, mxv)
    plsc.subcore_barrier()

    iota = lax.iota(jnp.int32, 16)
    mx_splat = [mxv[h] for h in range(4)]
    CW = EP // (KL * NW)

    def zpad(g, c):
        rows = g * 16 + iota
        zv = jnp.zeros((16,), jnp.float32)
        for cc in range(20, 24):
            plsc.store_scatter(ubuf, [rows, _i16(cc)], zv)
        return c

    lax.fori_loop(0, KL // 16, zpad, 0)

    def cbase(c):
        return (wid * CW + c) * KL

    def load_idx(c, sel):
        pltpu.sync_copy(src_h.at[pl.ds(cbase(c), KL)], sidx.at[sel])
        pltpu.sync_copy(dst_h.at[pl.ds(cbase(c), KL)], didx.at[sel])

    def fire(sel):
        pltpu.async_copy(s1_h.at[sidx.at[sel]], srows.at[sel], sem_a)
        pltpu.async_copy(d1_h.at[didx.at[sel]], drows.at[sel], sem_b)

    def drain(sel):
        pltpu.make_async_copy(s1_h.at[sidx.at[sel]], srows.at[sel], sem_a).wait()
        pltpu.make_async_copy(d1_h.at[didx.at[sel]], drows.at[sel], sem_b).wait()

    load_idx(0, 0)
    fire(0)

    def chunk(c, carry):
        sel = lax.rem(c, 2)
        nsel = 1 - sel
        base = cbase(c)
        drain(sel)

        @pl.when(c + 1 < CW)
        def _():
            load_idx(c + 1, nsel)
            fire(nsel)

        def group(g, cc):
            e0 = g * 16
            rows = e0 + iota
            sr = srows.at[sel]
            dr = drows.at[sel]
            asv = [plsc.load_gather(sr, [rows, _i16(h)]) for h in range(4)]
            xv = [plsc.load_gather(sr, [rows, _i16(4 + k)]) for k in range(4)]
            adv = [plsc.load_gather(dr, [rows, _i16(h)]) for h in range(4)]
            valid = (base + e0 + iota) < ER
            for h in range(4):
                a = _leaky(asv[h] + adv[h])
                cshift = _leaky(mx_splat[h] + adv[h])
                w = jnp.exp(a - cshift)
                w = jnp.where(valid, w, 0.0)
                plsc.store_scatter(ubuf, [rows, _i16(h)], w)
                for k in range(4):
                    plsc.store_scatter(ubuf, [rows, _i16(4 + h * 4 + k)], w * xv[k])
            return cc

        lax.fori_loop(0, KL // 16, group, 0)
        pltpu.sync_copy(ubuf, ush.at[didx.at[sel]], add=True)
        return carry

    lax.fori_loop(0, CW, chunk, 0)
    plsc.subcore_barrier()

    @pl.when(sid == 0)
    def _():
        pltpu.sync_copy(ush, up_h.at[cid])


def _l1_edges(s1, d1, srcp, dstp, mx1, zu):
    mesh = plsc.VectorSubcoreMesh(core_axis_name="c", subcore_axis_name="s")
    f = pl.kernel(
        _l1_body,
        out_type=jax.ShapeDtypeStruct((2, N, 24), jnp.float32),
        mesh=mesh,
        scratch_types=[
            pltpu.VMEM((2, KL), jnp.int32),
            pltpu.VMEM((2, KL), jnp.int32),
            pltpu.VMEM((2, KL, 8), jnp.float32),
            pltpu.VMEM((2, KL, 8), jnp.float32),
            pltpu.VMEM((KL, 24), jnp.float32),
            pltpu.VMEM((4, 16), jnp.float32),
            pltpu.VMEM_SHARED((N, 24), jnp.float32),
            pltpu.SemaphoreType.DMA,
            pltpu.SemaphoreType.DMA,
        ],
        compiler_params=pltpu.CompilerParams(**_SC_PARAMS),
    )
    return f(s1, d1, srcp, dstp, mx1, zu)


def _dense2_body(ua_ref, ub_ref, w1h0, w1h1, w1h2, w1h3, b1_ref, w2c_ref,
                 ats2_ref, atd2_ref, h2_ref, as2_ref, ad2_ref, mx2_ref):
    i = pl.program_id(0)
    u = ua_ref[...] + ub_ref[...]
    outs = []
    for h, w1h in enumerate((w1h0, w1h1, w1h2, w1h3)):
        den = u[:, h:h + 1]
        den = jnp.where(den > 0, den, 1.0)
        th = u[:, 4 + 4 * h:8 + 4 * h] / den
        outs.append(jnp.dot(th, w1h[...], preferred_element_type=jnp.float32))
    o1 = jnp.concatenate(outs, axis=1) + b1_ref[...]
    x1 = jnp.where(o1 > 0, o1, jnp.exp(o1) - 1.0)
    r2 = jnp.dot(x1, w2c_ref[...], preferred_element_type=jnp.float32)
    as2 = jnp.sum(r2 * ats2_ref[...], axis=1, keepdims=True)
    ad2 = jnp.sum(r2 * atd2_ref[...], axis=1, keepdims=True)
    h2_ref[...] = r2
    as2_ref[...] = as2
    ad2_ref[...] = ad2
    m = jnp.max(as2, axis=0, keepdims=True)

    @pl.when(i == 0)
    def _():
        mx2_ref[...] = m

    @pl.when(i > 0)
    def _():
        mx2_ref[...] = jnp.maximum(mx2_ref[...], m)


def _dense2(ua, ub, w1heads, b1r, w2c, ats2, atd2):
    B = 2000
    return pl.pallas_call(
        _dense2_body,
        grid=(N // B,),
        in_specs=[
            pl.BlockSpec((B, 24), lambda i: (i, 0)),
            pl.BlockSpec((B, 24), lambda i: (i, 0)),
            pl.BlockSpec((4, 32), lambda i: (0, 0)),
            pl.BlockSpec((4, 32), lambda i: (0, 0)),
            pl.BlockSpec((4, 32), lambda i: (0, 0)),
            pl.BlockSpec((4, 32), lambda i: (0, 0)),
            pl.BlockSpec((1, 128), lambda i: (0, 0)),
            pl.BlockSpec((128, 32), lambda i: (0, 0)),
            pl.BlockSpec((1, 32), lambda i: (0, 0)),
            pl.BlockSpec((1, 32), lambda i: (0, 0)),
        ],
        out_specs=[
            pl.BlockSpec((B, 32), lambda i: (i, 0)),
            pl.BlockSpec((B, 1), lambda i: (i, 0)),
            pl.BlockSpec((B, 1), lambda i: (i, 0)),
            pl.BlockSpec((1, 1), lambda i: (0, 0)),
        ],
        out_shape=[
            jax.ShapeDtypeStruct((N, 32), jnp.float32),
            jax.ShapeDtypeStruct((N, 1), jnp.float32),
            jax.ShapeDtypeStruct((N, 1), jnp.float32),
            jax.ShapeDtypeStruct((1, 1), jnp.float32),
        ],
    )(ua, ub, *w1heads, b1r, w2c, ats2, atd2)


def _pc_body(src_h, dst_h, as2_h, ad2_h, mx2_h, zn_h, w2_h, dena_h, denb_h,
             sidx, didx, asv, adv, wbuf, mxv, dsh, sem_a, sem_b):
    cid = lax.axis_index("c")
    sid = lax.axis_index("s")
    wid = cid * 16 + sid

    @pl.when(sid == 0)
    def _():
        pltpu.sync_copy(zn_h, dsh)

    pltpu.sync_copy(mx2_h, mxv)
    plsc.subcore_barrier()

    iota = lax.iota(jnp.int32, 16)
    mxs = mxv[...]
    CW = EP // (KC * NW)

    def cbase(c):
        return (wid * CW + c) * KC

    def load_idx(c, sel):
        pltpu.sync_copy(src_h.at[pl.ds(cbase(c), KC)], sidx.at[sel])
        pltpu.sync_copy(dst_h.at[pl.ds(cbase(c), KC)], didx.at[sel])

    def fire(sel):
        pltpu.async_copy(as2_h.at[sidx.at[sel]], asv.at[sel], sem_a)
        pltpu.async_copy(ad2_h.at[didx.at[sel]], adv.at[sel], sem_b)

    def drain(sel):
        pltpu.make_async_copy(as2_h.at[sidx.at[sel]], asv.at[sel], sem_a).wait()
        pltpu.make_async_copy(ad2_h.at[didx.at[sel]], adv.at[sel], sem_b).wait()

    load_idx(0, 0)
    fire(0)

    def chunk(c, carry):
        sel = lax.rem(c, 2)
        nsel = 1 - sel
        base = cbase(c)
        drain(sel)

        @pl.when(c + 1 < CW)
        def _():
            load_idx(c + 1, nsel)
            fire(nsel)

        def group(g, cc):
            rows = g * 16 + iota
            a_s = plsc.load_gather(asv.at[sel], [rows])
            a_d = plsc.load_gather(adv.at[sel], [rows])
            a = _leaky(a_s + a_d)
            cshift = _leaky(mxs + a_d)
            w = jnp.exp(a - cshift)
            valid = (base + g * 16 + iota) < ER
            w = jnp.where(valid, w, 0.0)
            plsc.store_scatter(wbuf, [rows], w)
            return cc

        lax.fori_loop(0, KC // 16, group, 0)
        pltpu.sync_copy(wbuf, w2_h.at[pl.ds(base, KC)])
        pltpu.sync_copy(wbuf, dsh.at[didx.at[sel]], add=True)
        return carry

    lax.fori_loop(0, CW, chunk, 0)
    plsc.subcore_barrier()

    @pl.when(jnp.logical_and(sid == 0, cid == 0))
    def _():
        pltpu.sync_copy(dsh, dena_h)

    @pl.when(jnp.logical_and(sid == 0, cid == 1))
    def _():
        pltpu.sync_copy(dsh, denb_h)


def _pass_c(srcp, dstp, as2, ad2, mx2, zn):
    mesh = plsc.VectorSubcoreMesh(core_axis_name="c", subcore_axis_name="s")
    f = pl.kernel(
        _pc_body,
        out_type=(
            jax.ShapeDtypeStruct((EP,), jnp.float32),
            jax.ShapeDtypeStruct((N,), jnp.float32),
            jax.ShapeDtypeStruct((N,), jnp.float32),
        ),
        mesh=mesh,
        scratch_types=[
            pltpu.VMEM((2, KC), jnp.int32),
            pltpu.VMEM((2, KC), jnp.int32),
            pltpu.VMEM((2, KC), jnp.float32),
            pltpu.VMEM((2, KC), jnp.float32),
            pltpu.VMEM((KC,), jnp.float32),
            pltpu.VMEM((16,), jnp.float32),
            pltpu.VMEM_SHARED((N,), jnp.float32),
            pltpu.SemaphoreType.DMA,
            pltpu.SemaphoreType.DMA,
        ],
        compiler_params=pltpu.CompilerParams(**_SC_PARAMS),
    )
    return f(srcp, dstp, as2, ad2, mx2, zn)


def _pd_body(src_h, dst_h, w2_h, dena_h, denb_h, h2_h, zo_h, attn_h, o2_h,
             sidx, didx, wv, dav, dbv, hrows, attnb, osh,
             sem_a, sem_b, sem_c, sem_w):
    cid = lax.axis_index("c")
    sid = lax.axis_index("s")
    wid = cid * 16 + sid

    @pl.when(sid == 0)
    def _():
        pltpu.sync_copy(zo_h, osh)

    plsc.subcore_barrier()

    iota = lax.iota(jnp.int32, 16)
    CW = EP // (KD * NW)

    def cbase(c):
        return (wid * CW + c) * KD

    def load_idx(c, sel):
        pltpu.sync_copy(src_h.at[pl.ds(cbase(c), KD)], sidx.at[sel])
        pltpu.sync_copy(dst_h.at[pl.ds(cbase(c), KD)], didx.at[sel])

    def fire(c, sel):
        pltpu.async_copy(w2_h.at[pl.ds(cbase(c), KD)], wv.at[sel], sem_w)
        pltpu.async_copy(dena_h.at[didx.at[sel]], dav.at[sel], sem_a)
        pltpu.async_copy(denb_h.at[didx.at[sel]], dbv.at[sel], sem_b)
        pltpu.async_copy(h2_h.at[sidx.at[sel]], hrows.at[sel], sem_c)

    def drain(c, sel):
        pltpu.make_async_copy(w2_h.at[pl.ds(cbase(c), KD)], wv.at[sel], sem_w).wait()
        pltpu.make_async_copy(dena_h.at[didx.at[sel]], dav.at[sel], sem_a).wait()
        pltpu.make_async_copy(denb_h.at[didx.at[sel]], dbv.at[sel], sem_b).wait()
        pltpu.make_async_copy(h2_h.at[sidx.at[sel]], hrows.at[sel], sem_c).wait()

    load_idx(0, 0)
    fire(0, 0)

    def chunk(c, carry):
        sel = lax.rem(c, 2)
        nsel = 1 - sel
        base = cbase(c)
        drain(c, sel)

        @pl.when(c + 1 < CW)
        def _():
            load_idx(c + 1, nsel)
            fire(c + 1, nsel)

        hr = hrows.at[sel]

        def group(g, cc):
            rows = g * 16 + iota
            w = plsc.load_gather(wv.at[sel], [rows])
            da = plsc.load_gather(dav.at[sel], [rows])
            db = plsc.load_gather(dbv.at[sel], [rows])
            att = w / (da + db + 1e-16)
            plsc.store_scatter(attnb, [rows], att)
            for col in range(32):
                hv = plsc.load_gather(hr, [rows, _i16(col)])
                plsc.store_scatter(hr, [rows, _i16(col)], hv * att)
            return cc

        lax.fori_loop(0, KD // 16, group, 0)
        pltpu.sync_copy(attnb, attn_h.at[pl.ds(base, KD)])
        pltpu.sync_copy(hrows.at[sel], osh.at[didx.at[sel]], add=True)
        return carry

    lax.fori_loop(0, CW, chunk, 0)
    plsc.subcore_barrier()

    @pl.when(sid == 0)
    def _():
        pltpu.sync_copy(osh, o2_h.at[cid])


def _pass_d(srcp, dstp, w2, dena, denb, h2, zo):
    mesh = plsc.VectorSubcoreMesh(core_axis_name="c", subcore_axis_name="s")
    f = pl.kernel(
        _pd_body,
        out_type=(
            jax.ShapeDtypeStruct((EP,), jnp.float32),
            jax.ShapeDtypeStruct((2, N, 32), jnp.float32),
        ),
        mesh=mesh,
        scratch_types=[
            pltpu.VMEM((2, KD), jnp.int32),
            pltpu.VMEM((2, KD), jnp.int32),
            pltpu.VMEM((2, KD), jnp.float32),
            pltpu.VMEM((2, KD), jnp.float32),
            pltpu.VMEM((2, KD), jnp.float32),
            pltpu.VMEM((2, KD, 32), jnp.float32),
            pltpu.VMEM((KD,), jnp.float32),
            pltpu.VMEM_SHARED((N, 32), jnp.float32),
            pltpu.SemaphoreType.DMA,
            pltpu.SemaphoreType.DMA,
            pltpu.SemaphoreType.DMA,
            pltpu.SemaphoreType.DMA,
        ],
        compiler_params=pltpu.CompilerParams(**_SC_PARAMS),
    )
    return f(srcp, dstp, w2, dena, denb, h2, zo)


def _dense3_body(oa_ref, ob_ref, b2_ref, wl_ref, bl_ref, sum_ref, log_ref):
    i = pl.program_id(0)
    o = oa_ref[...] + ob_ref[...] + b2_ref[...]
    x2 = jnp.where(o > 0, o, jnp.exp(o) - 1.0)
    s = jnp.sum(x2, axis=0, keepdims=True)

    @pl.when(i == 0)
    def _():
        sum_ref[...] = s

    @pl.when(i > 0)
    def _():
        sum_ref[...] = sum_ref[...] + s

    @pl.when(i == pl.num_programs(0) - 1)
    def _():
        log_ref[...] = (
            jnp.dot(sum_ref[...] * (1.0 / N), wl_ref[...],
                    preferred_element_type=jnp.float32) + bl_ref[...])


def _dense3(oa, ob, b2r, wl, blr):
    B = 2000
    return pl.pallas_call(
        _dense3_body,
        grid=(N // B,),
        in_specs=[
            pl.BlockSpec((B, 32), lambda i: (i, 0)),
            pl.BlockSpec((B, 32), lambda i: (i, 0)),
            pl.BlockSpec((1, 32), lambda i: (0, 0)),
            pl.BlockSpec((32, 2), lambda i: (0, 0)),
            pl.BlockSpec((1, 2), lambda i: (0, 0)),
        ],
        out_specs=[
            pl.BlockSpec((1, 32), lambda i: (0, 0)),
            pl.BlockSpec((1, 2), lambda i: (0, 0)),
        ],
        out_shape=[
            jax.ShapeDtypeStruct((1, 32), jnp.float32),
            jax.ShapeDtypeStruct((1, 2), jnp.float32),
        ],
    )(oa, ob, b2r, wl, blr)


def kernel(x, edge_index, W1, a_s1, a_d1, b1, W2, a_s2, a_d2, b2, Wl, bl):
    loops = jnp.arange(N, dtype=jnp.int32)
    padz = jnp.zeros((EP - ER,), jnp.int32)
    srcp = jnp.concatenate([edge_index[0].astype(jnp.int32), loops, padz])
    dstp = jnp.concatenate([edge_index[1].astype(jnp.int32), loops, padz])

    w1r = W1.reshape(4, 4, 32)
    w1heads = [w1r[:, h, :] for h in range(4)]

    zu = jnp.zeros((N, 24), jnp.float32)
    zn = jnp.zeros((N,), jnp.float32)
    zo = jnp.zeros((N, 32), jnp.float32)

    s1, d1t, mx1 = _dense1(x, W1, a_s1, a_d1)
    mxtab = jnp.broadcast_to(mx1.reshape(16)[:4][:, None], (4, 16))
    up = _l1_edges(s1, d1t, srcp, dstp, mxtab, zu)
    h2, as2, ad2, mx2 = _dense2(up[0], up[1], w1heads, b1.reshape(1, 128), W2,
                                a_s2.reshape(1, 32), a_d2.reshape(1, 32))
    mx2v = jnp.broadcast_to(mx2.reshape(1), (16,))
    w2e, dena, denb = _pass_c(srcp, dstp, as2.reshape(N), ad2.reshape(N),
                              mx2v, zn)
    attnp, o2 = _pass_d(srcp, dstp, w2e, dena, denb, h2, zo)
    _, logits = _dense3(o2[0], o2[1], b2.reshape(1, 32), Wl, bl.reshape(1, 2))
    attn = attnp[:ER].reshape(ER, 1)
    return logits, attn


# speedup vs baseline: 124.8383x; 1.8779x over previous
import jax
import jax.numpy as jnp
from jax import lax
from jax.experimental import pallas as pl
from jax.experimental.pallas import tpu as pltpu
from jax.experimental.pallas import tpu_sc as plsc

N = 50000
E = 800000
ER = E + N
NW = 32
KL = 512
KC = 1024
KD = 1024
EP = 1024 * NW * (-(-ER // (1024 * NW)))
_SC_PARAMS = dict(needs_layout_passes=False, use_tc_tiling_on_sc=False)


def _i16(v):
    return jnp.full((16,), v, jnp.int32)


def _leaky(v):
    return jnp.where(v > 0, v, 0.2 * v)


def _dense1_body(x_ref, w1_ref, ats_ref, atd_ref, s1_ref, d1_ref, mx_ref):
    i = pl.program_id(0)
    xb = x_ref[...]
    h1 = jnp.dot(xb, w1_ref[...], preferred_element_type=jnp.float32)
    acols, dcols = [], []
    for h in range(4):
        hh = h1[:, 32 * h:32 * h + 32]
        acols.append(jnp.sum(hh * ats_ref[h:h + 1, :], axis=1, keepdims=True))
        dcols.append(jnp.sum(hh * atd_ref[h:h + 1, :], axis=1, keepdims=True))
    ts = jnp.concatenate(acols + [xb], axis=1)
    td = jnp.concatenate(dcols + [jnp.zeros_like(xb)], axis=1)
    s1_ref[...] = ts
    d1_ref[...] = td
    m8 = jnp.max(ts[:, :8], axis=0, keepdims=True)
    m = jnp.concatenate([m8, m8], axis=1)

    @pl.when(i == 0)
    def _():
        mx_ref[...] = m

    @pl.when(i > 0)
    def _():
        mx_ref[...] = jnp.maximum(mx_ref[...], m)


def _dense1(x, w1, ats, atd):
    B = 5000
    return pl.pallas_call(
        _dense1_body,
        grid=(N // B,),
        in_specs=[
            pl.BlockSpec((B, 4), lambda i: (i, 0)),
            pl.BlockSpec((4, 128), lambda i: (0, 0)),
            pl.BlockSpec((4, 32), lambda i: (0, 0)),
            pl.BlockSpec((4, 32), lambda i: (0, 0)),
        ],
        out_specs=[
            pl.BlockSpec((B, 8), lambda i: (i, 0)),
            pl.BlockSpec((B, 8), lambda i: (i, 0)),
            pl.BlockSpec((1, 16), lambda i: (0, 0)),
        ],
        out_shape=[
            jax.ShapeDtypeStruct((N, 8), jnp.float32),
            jax.ShapeDtypeStruct((N, 8), jnp.float32),
            jax.ShapeDtypeStruct((1, 16), jnp.float32),
        ],
    )(x, w1, ats, atd)


def _l1_body(s1_h, d1_h, src_h, dst_h, mx_h, zu_h, up_h,
             sidx, didx, srows, drows, ubuf, mxv, ush, sem_a, sem_b):
    cid = lax.axis_index("c")
    sid = lax.axis_index("s")
    wid = cid * 16 + sid

    @pl.when(sid == 0)
    def _():
        pltpu.sync_copy(zu_h, ush)

    pltpu.sync_copy(mx_h, mxv)
    plsc.subcore_barrier()

    iota = lax.iota(jnp.int32, 16)
    mx_splat = [mxv[h] for h in range(4)]
    CW = EP // (KL * NW)

    def zpad(g, c):
        rows = g * 16 + iota
        zv = jnp.zeros((16,), jnp.float32)
        for cc in range(20, 24):
            plsc.store_scatter(ubuf, [rows, _i16(cc)], zv)
        return c

    lax.fori_loop(0, KL // 16, zpad, 0)

    def cbase(c):
        return (wid * CW + c) * KL

    def load_idx(c, sel):
        pltpu.sync_copy(src_h.at[pl.ds(cbase(c), KL)], sidx.at[sel])
        pltpu.sync_copy(dst_h.at[pl.ds(cbase(c), KL)], didx.at[sel])

    def fire(sel):
        pltpu.async_copy(s1_h.at[sidx.at[sel]], srows.at[sel], sem_a)
        pltpu.async_copy(d1_h.at[didx.at[sel]], drows.at[sel], sem_b)

    def drain(sel):
        pltpu.make_async_copy(s1_h.at[sidx.at[sel]], srows.at[sel], sem_a).wait()
        pltpu.make_async_copy(d1_h.at[didx.at[sel]], drows.at[sel], sem_b).wait()

    load_idx(0, 0)
    fire(0)

    def chunk(c, carry):
        sel = lax.rem(c, 2)
        nsel = 1 - sel
        base = cbase(c)
        drain(sel)

        @pl.when(c + 1 < CW)
        def _():
            load_idx(c + 1, nsel)
            fire(nsel)

        def group(g, cc):
            e0 = g * 16
            rows = e0 + iota
            sr = srows.at[sel]
            dr = drows.at[sel]
            asv = [plsc.load_gather(sr, [rows, _i16(h)]) for h in range(4)]
            xv = [plsc.load_gather(sr, [rows, _i16(4 + k)]) for k in range(4)]
            adv = [plsc.load_gather(dr, [rows, _i16(h)]) for h in range(4)]
            valid = (base + e0 + iota) < ER
            for h in range(4):
                a = _leaky(asv[h] + adv[h])
                cshift = _leaky(mx_splat[h] + adv[h])
                w = jnp.exp(a - cshift)
                w = jnp.where(valid, w, 0.0)
                plsc.store_scatter(ubuf, [rows, _i16(h)], w)
                for k in range(4):
                    plsc.store_scatter(ubuf, [rows, _i16(4 + h * 4 + k)], w * xv[k])
            return cc

        lax.fori_loop(0, KL // 16, group, 0)
        pltpu.sync_copy(ubuf, ush.at[didx.at[sel]], add=True)
        return carry

    lax.fori_loop(0, CW, chunk, 0)
    plsc.subcore_barrier()

    @pl.when(sid == 0)
    def _():
        pltpu.sync_copy(ush, up_h.at[cid])


def _l1_edges(s1, d1, srcp, dstp, mx1, zu):
    mesh = plsc.VectorSubcoreMesh(core_axis_name="c", subcore_axis_name="s")
    f = pl.kernel(
        _l1_body,
        out_type=jax.ShapeDtypeStruct((2, N, 24), jnp.float32),
        mesh=mesh,
        scratch_types=[
            pltpu.VMEM((2, KL), jnp.int32),
            pltpu.VMEM((2, KL), jnp.int32),
            pltpu.VMEM((2, KL, 8), jnp.float32),
            pltpu.VMEM((2, KL, 8), jnp.float32),
            pltpu.VMEM((KL, 24), jnp.float32),
            pltpu.VMEM((4, 16), jnp.float32),
            pltpu.VMEM_SHARED((N, 24), jnp.float32),
            pltpu.SemaphoreType.DMA,
            pltpu.SemaphoreType.DMA,
        ],
        compiler_params=pltpu.CompilerParams(**_SC_PARAMS),
    )
    return f(s1, d1, srcp, dstp, mx1, zu)


def _dense2_body(ua_ref, ub_ref, w1h0, w1h1, w1h2, w1h3, b1_ref, w2c_ref,
                 ats2_ref, atd2_ref, h2_ref, as2_ref, ad2_ref, mx2_ref):
    i = pl.program_id(0)
    u = ua_ref[...] + ub_ref[...]
    outs = []
    for h, w1h in enumerate((w1h0, w1h1, w1h2, w1h3)):
        den = u[:, h:h + 1]
        den = jnp.where(den > 0, den, 1.0)
        th = u[:, 4 + 4 * h:8 + 4 * h] / den
        outs.append(jnp.dot(th, w1h[...], preferred_element_type=jnp.float32))
    o1 = jnp.concatenate(outs, axis=1) + b1_ref[...]
    x1 = jnp.where(o1 > 0, o1, jnp.exp(o1) - 1.0)
    r2 = jnp.dot(x1, w2c_ref[...], preferred_element_type=jnp.float32)
    as2 = jnp.sum(r2 * ats2_ref[...], axis=1, keepdims=True)
    ad2 = jnp.sum(r2 * atd2_ref[...], axis=1, keepdims=True)
    h2_ref[...] = r2
    as2_ref[...] = as2
    ad2_ref[...] = ad2
    m = jnp.max(as2, axis=0, keepdims=True)

    @pl.when(i == 0)
    def _():
        mx2_ref[...] = m

    @pl.when(i > 0)
    def _():
        mx2_ref[...] = jnp.maximum(mx2_ref[...], m)


def _dense2(ua, ub, w1heads, b1r, w2c, ats2, atd2):
    B = 2000
    return pl.pallas_call(
        _dense2_body,
        grid=(N // B,),
        in_specs=[
            pl.BlockSpec((B, 24), lambda i: (i, 0)),
            pl.BlockSpec((B, 24), lambda i: (i, 0)),
            pl.BlockSpec((4, 32), lambda i: (0, 0)),
            pl.BlockSpec((4, 32), lambda i: (0, 0)),
            pl.BlockSpec((4, 32), lambda i: (0, 0)),
            pl.BlockSpec((4, 32), lambda i: (0, 0)),
            pl.BlockSpec((1, 128), lambda i: (0, 0)),
            pl.BlockSpec((128, 32), lambda i: (0, 0)),
            pl.BlockSpec((1, 32), lambda i: (0, 0)),
            pl.BlockSpec((1, 32), lambda i: (0, 0)),
        ],
        out_specs=[
            pl.BlockSpec((B, 32), lambda i: (i, 0)),
            pl.BlockSpec((B, 1), lambda i: (i, 0)),
            pl.BlockSpec((B, 1), lambda i: (i, 0)),
            pl.BlockSpec((1, 1), lambda i: (0, 0)),
        ],
        out_shape=[
            jax.ShapeDtypeStruct((N, 32), jnp.float32),
            jax.ShapeDtypeStruct((N, 1), jnp.float32),
            jax.ShapeDtypeStruct((N, 1), jnp.float32),
            jax.ShapeDtypeStruct((1, 1), jnp.float32),
        ],
    )(ua, ub, *w1heads, b1r, w2c, ats2, atd2)


def _pc_body(src_h, dst_h, as2_h, ad2_h, mx2_h, zn_h, w2_h, dena_h, denb_h,
             sidx, didx, asv, adv, wbuf, mxv, dsh, sem_a, sem_b):
    cid = lax.axis_index("c")
    sid = lax.axis_index("s")
    wid = cid * 16 + sid

    @pl.when(sid == 0)
    def _():
        pltpu.sync_copy(zn_h, dsh)

    pltpu.sync_copy(mx2_h, mxv)
    plsc.subcore_barrier()

    iota = lax.iota(jnp.int32, 16)
    mxs = mxv[...]
    CW = EP // (KC * NW)

    def cbase(c):
        return (wid * CW + c) * KC

    def load_idx(c, sel):
        pltpu.sync_copy(src_h.at[pl.ds(cbase(c), KC)], sidx.at[sel])
        pltpu.sync_copy(dst_h.at[pl.ds(cbase(c), KC)], didx.at[sel])

    def fire(sel):
        pltpu.async_copy(as2_h.at[sidx.at[sel]], asv.at[sel], sem_a)
        pltpu.async_copy(ad2_h.at[didx.at[sel]], adv.at[sel], sem_b)

    def drain(sel):
        pltpu.make_async_copy(as2_h.at[sidx.at[sel]], asv.at[sel], sem_a).wait()
        pltpu.make_async_copy(ad2_h.at[didx.at[sel]], adv.at[sel], sem_b).wait()

    load_idx(0, 0)
    fire(0)

    def chunk(c, carry):
        sel = lax.rem(c, 2)
        nsel = 1 - sel
        base = cbase(c)
        drain(sel)

        @pl.when(c + 1 < CW)
        def _():
            load_idx(c + 1, nsel)
            fire(nsel)

        def group(g, cc):
            rows = g * 16 + iota
            a_s = plsc.load_gather(asv.at[sel], [rows])
            a_d = plsc.load_gather(adv.at[sel], [rows])
            a = _leaky(a_s + a_d)
            cshift = _leaky(mxs + a_d)
            w = jnp.exp(a - cshift)
            valid = (base + g * 16 + iota) < ER
            w = jnp.where(valid, w, 0.0)
            plsc.store_scatter(wbuf, [rows], w)
            return cc

        lax.fori_loop(0, KC // 16, group, 0)
        pltpu.sync_copy(wbuf, w2_h.at[pl.ds(base, KC)])
        pltpu.sync_copy(wbuf, dsh.at[didx.at[sel]], add=True)
        return carry

    lax.fori_loop(0, CW, chunk, 0)
    plsc.subcore_barrier()

    @pl.when(jnp.logical_and(sid == 0, cid == 0))
    def _():
        pltpu.sync_copy(dsh, dena_h)

    @pl.when(jnp.logical_and(sid == 0, cid == 1))
    def _():
        pltpu.sync_copy(dsh, denb_h)


def _pass_c(srcp, dstp, as2, ad2, mx2, zn):
    mesh = plsc.VectorSubcoreMesh(core_axis_name="c", subcore_axis_name="s")
    f = pl.kernel(
        _pc_body,
        out_type=(
            jax.ShapeDtypeStruct((EP,), jnp.float32),
            jax.ShapeDtypeStruct((N,), jnp.float32),
            jax.ShapeDtypeStruct((N,), jnp.float32),
        ),
        mesh=mesh,
        scratch_types=[
            pltpu.VMEM((2, KC), jnp.int32),
            pltpu.VMEM((2, KC), jnp.int32),
            pltpu.VMEM((2, KC), jnp.float32),
            pltpu.VMEM((2, KC), jnp.float32),
            pltpu.VMEM((KC,), jnp.float32),
            pltpu.VMEM((16,), jnp.float32),
            pltpu.VMEM_SHARED((N,), jnp.float32),
            pltpu.SemaphoreType.DMA,
            pltpu.SemaphoreType.DMA,
        ],
        compiler_params=pltpu.CompilerParams(**_SC_PARAMS),
    )
    return f(srcp, dstp, as2, ad2, mx2, zn)


def _pd_body(src_h, dst_h, w2_h, dena_h, denb_h, h2_h, zo_h, attn_h, o2_h,
             sidx, didx, wv, dav, dbv, hrows, hbf, attnb, osh,
             sem_a, sem_b, sem_c, sem_w):
    cid = lax.axis_index("c")
    sid = lax.axis_index("s")
    wid = cid * 16 + sid

    @pl.when(sid == 0)
    def _():
        pltpu.sync_copy(zo_h, osh)

    plsc.subcore_barrier()

    iota = lax.iota(jnp.int32, 16)
    CW = EP // (KD * NW)

    def cbase(c):
        return (wid * CW + c) * KD

    def load_idx(c, sel):
        pltpu.sync_copy(src_h.at[pl.ds(cbase(c), KD)], sidx.at[sel])
        pltpu.sync_copy(dst_h.at[pl.ds(cbase(c), KD)], didx.at[sel])

    def fire(c, sel):
        pltpu.async_copy(w2_h.at[pl.ds(cbase(c), KD)], wv.at[sel], sem_w)
        pltpu.async_copy(dena_h.at[didx.at[sel]], dav.at[sel], sem_a)
        pltpu.async_copy(denb_h.at[didx.at[sel]], dbv.at[sel], sem_b)
        pltpu.async_copy(h2_h.at[sidx.at[sel]], hrows.at[sel], sem_c)

    def drain(c, sel):
        pltpu.make_async_copy(w2_h.at[pl.ds(cbase(c), KD)], wv.at[sel], sem_w).wait()
        pltpu.make_async_copy(dena_h.at[didx.at[sel]], dav.at[sel], sem_a).wait()
        pltpu.make_async_copy(denb_h.at[didx.at[sel]], dbv.at[sel], sem_b).wait()
        pltpu.make_async_copy(h2_h.at[sidx.at[sel]], hrows.at[sel], sem_c).wait()

    load_idx(0, 0)
    fire(0, 0)

    def chunk(c, carry):
        sel = lax.rem(c, 2)
        nsel = 1 - sel
        base = cbase(c)
        drain(c, sel)

        @pl.when(c + 1 < CW)
        def _():
            load_idx(c + 1, nsel)
            fire(c + 1, nsel)

        hr = hrows.at[sel]

        def group(g, cc):
            rows = g * 16 + iota
            w = plsc.load_gather(wv.at[sel], [rows])
            da = plsc.load_gather(dav.at[sel], [rows])
            db = plsc.load_gather(dbv.at[sel], [rows])
            att = w / (da + db + 1e-16)
            plsc.store_scatter(attnb, [rows], att)
            for l in range(16):
                e = g * 16 + l
                row = hr[e, :]
                ua, ub = plsc.unpack(row, format=plsc.PackFormat.INTERLEAVED)
                spl = jnp.zeros((16,), jnp.float32) + att[l]
                packed = plsc.pack(ua * spl, ub * spl,
                                   format=plsc.PackFormat.INTERLEAVED)
                hbf[e, :] = packed
            return cc

        lax.fori_loop(0, KD // 16, group, 0)
        pltpu.sync_copy(attnb, attn_h.at[pl.ds(base, KD)])
        pltpu.sync_copy(hbf, osh.at[didx.at[sel]], add=True)
        return carry

    lax.fori_loop(0, CW, chunk, 0)
    plsc.subcore_barrier()

    @pl.when(sid == 0)
    def _():
        pltpu.sync_copy(osh, o2_h.at[cid])


def _pass_d(srcp, dstp, w2, dena, denb, h2, zo):
    mesh = plsc.VectorSubcoreMesh(core_axis_name="c", subcore_axis_name="s")
    f = pl.kernel(
        _pd_body,
        out_type=(
            jax.ShapeDtypeStruct((EP,), jnp.float32),
            jax.ShapeDtypeStruct((2, N, 32), jnp.bfloat16),
        ),
        mesh=mesh,
        scratch_types=[
            pltpu.VMEM((2, KD), jnp.int32),
            pltpu.VMEM((2, KD), jnp.int32),
            pltpu.VMEM((2, KD), jnp.float32),
            pltpu.VMEM((2, KD), jnp.float32),
            pltpu.VMEM((2, KD), jnp.float32),
            pltpu.VMEM((2, KD, 32), jnp.bfloat16),
            pltpu.VMEM((KD, 32), jnp.bfloat16),
            pltpu.VMEM((KD,), jnp.float32),
            pltpu.VMEM_SHARED((N, 32), jnp.bfloat16),
            pltpu.SemaphoreType.DMA,
            pltpu.SemaphoreType.DMA,
            pltpu.SemaphoreType.DMA,
            pltpu.SemaphoreType.DMA,
        ],
        compiler_params=pltpu.CompilerParams(**_SC_PARAMS),
    )
    return f(srcp, dstp, w2, dena, denb, h2, zo)


def _dense3_body(oa_ref, ob_ref, b2_ref, wl_ref, bl_ref, sum_ref, log_ref):
    i = pl.program_id(0)
    o = (oa_ref[...].astype(jnp.float32) + ob_ref[...].astype(jnp.float32)
         + b2_ref[...])
    x2 = jnp.where(o > 0, o, jnp.exp(o) - 1.0)
    s = jnp.sum(x2, axis=0, keepdims=True)

    @pl.when(i == 0)
    def _():
        sum_ref[...] = s

    @pl.when(i > 0)
    def _():
        sum_ref[...] = sum_ref[...] + s

    @pl.when(i == pl.num_programs(0) - 1)
    def _():
        log_ref[...] = (
            jnp.dot(sum_ref[...] * (1.0 / N), wl_ref[...],
                    preferred_element_type=jnp.float32) + bl_ref[...])


def _dense3(oa, ob, b2r, wl, blr):
    B = 2000
    return pl.pallas_call(
        _dense3_body,
        grid=(N // B,),
        in_specs=[
            pl.BlockSpec((B, 32), lambda i: (i, 0)),
            pl.BlockSpec((B, 32), lambda i: (i, 0)),
            pl.BlockSpec((1, 32), lambda i: (0, 0)),
            pl.BlockSpec((32, 2), lambda i: (0, 0)),
            pl.BlockSpec((1, 2), lambda i: (0, 0)),
        ],
        out_specs=[
            pl.BlockSpec((1, 32), lambda i: (0, 0)),
            pl.BlockSpec((1, 2), lambda i: (0, 0)),
        ],
        out_shape=[
            jax.ShapeDtypeStruct((1, 32), jnp.float32),
            jax.ShapeDtypeStruct((1, 2), jnp.float32),
        ],
    )(oa, ob, b2r, wl, blr)


def kernel(x, edge_index, W1, a_s1, a_d1, b1, W2, a_s2, a_d2, b2, Wl, bl):
    loops = jnp.arange(N, dtype=jnp.int32)
    padz = jnp.zeros((EP - ER,), jnp.int32)
    srcp = jnp.concatenate([edge_index[0].astype(jnp.int32), loops, padz])
    dstp = jnp.concatenate([edge_index[1].astype(jnp.int32), loops, padz])

    w1r = W1.reshape(4, 4, 32)
    w1heads = [w1r[:, h, :] for h in range(4)]

    zu = jnp.zeros((N, 24), jnp.float32)
    zn = jnp.zeros((N,), jnp.float32)
    zo = jnp.zeros((N, 32), jnp.bfloat16)
    perm = jnp.stack([jnp.arange(16, dtype=jnp.int32),
                      jnp.arange(16, dtype=jnp.int32) + 16], axis=1).reshape(32)

    s1, d1t, mx1 = _dense1(x, W1, a_s1, a_d1)
    mxtab = jnp.broadcast_to(mx1.reshape(16)[:4][:, None], (4, 16))
    up = _l1_edges(s1, d1t, srcp, dstp, mxtab, zu)
    h2, as2, ad2, mx2 = _dense2(up[0], up[1], w1heads, b1.reshape(1, 128), W2,
                                a_s2.reshape(1, 32), a_d2.reshape(1, 32))
    mx2v = jnp.broadcast_to(mx2.reshape(1), (16,))
    w2e, dena, denb = _pass_c(srcp, dstp, as2.reshape(N), ad2.reshape(N),
                              mx2v, zn)
    h2bf = h2[:, perm].astype(jnp.bfloat16)
    attnp, o2 = _pass_d(srcp, dstp, w2e, dena, denb, h2bf, zo)
    _, logits = _dense3(o2[0], o2[1], b2[perm].reshape(1, 32), Wl[perm, :],
                        bl.reshape(1, 2))
    attn = attnp[:ER].reshape(ER, 1)
    return logits, attn
